# Initial kernel scaffold; baseline (speedup 1.0000x reference)
import jax
import jax.numpy as jnp
from jax.experimental import pallas as pl


def kernel(pos, pos_embedding):
    raise NotImplementedError("write your pallas kernel here")


# speedup vs baseline: 2.3763x; 2.3763x over previous
import functools

import jax
import jax.numpy as jnp
from jax import lax
from jax.experimental import pallas as pl
from jax.experimental.pallas import tpu as pltpu
from jax.experimental.pallas import tpu_sc as plsc


_NC, _NS = 2, 16
_NW = _NC * _NS

_CHUNK = 32
_NBUF = 2


@functools.partial(jax.jit, static_argnames=("b", "d"))
def _sc_gather(table, idx, *, b, d):
    b_per_w = b // _NW
    nch = b_per_w // _CHUNK
    mesh = plsc.VectorSubcoreMesh(core_axis_name="c", subcore_axis_name="s")

    @functools.partial(
        pl.kernel,
        mesh=mesh,
        out_type=jax.ShapeDtypeStruct((b, d), jnp.float32),
        scratch_types=[
            pltpu.VMEM((b_per_w,), jnp.int32),
            pltpu.VMEM((_CHUNK, d), jnp.float32),
            pltpu.VMEM((_CHUNK, d), jnp.float32),
            pltpu.SemaphoreType.DMA,
            pltpu.SemaphoreType.DMA,
            pltpu.SemaphoreType.DMA,
            pltpu.SemaphoreType.DMA,
        ],
    )
    def k(table_hbm, idx_hbm, out_hbm, idx_v, buf0, buf1, g0, g1, s0, s1):
        wid = lax.axis_index("s") * _NC + lax.axis_index("c")
        base = pl.multiple_of(wid * b_per_w, 8)
        pltpu.sync_copy(idx_hbm.at[pl.ds(base, b_per_w)], idx_v)

        bufs = (buf0, buf1)
        gsems = (g0, g1)
        ssems = (s0, s1)

        def gather_start(slot, ch):
            off = pl.multiple_of(ch * _CHUNK, 8)
            pltpu.async_copy(
                table_hbm.at[idx_v.at[pl.ds(off, _CHUNK)]],
                bufs[slot],
                gsems[slot],
            )

        for slot in range(_NBUF):
            gather_start(slot, slot)

        def step(i, carry):
            for slot in range(_NBUF):
                ch = i * _NBUF + slot
                pltpu.make_async_copy(
                    table_hbm.at[pl.ds(0, _CHUNK)], bufs[slot], gsems[slot]
                ).wait()
                row = pl.multiple_of(base + ch * _CHUNK, 8)
                pltpu.async_copy(
                    bufs[slot], out_hbm.at[pl.ds(row, _CHUNK)], ssems[slot]
                )
                pltpu.make_async_copy(
                    bufs[slot], out_hbm.at[pl.ds(base, _CHUNK)], ssems[slot]
                ).wait()
                nxt = ch + _NBUF

                @pl.when(nxt < nch)
                def _():
                    gather_start(slot, nxt)

            return carry

        lax.fori_loop(0, nch // _NBUF, step, 0)

    return k(table, idx)


def kernel(pos, pos_embedding):
    b = pos.size
    d = pos_embedding.shape[1]
    idx = pos.reshape(b).astype(jnp.int32)
    out = _sc_gather(pos_embedding, idx, b=b, d=d)
    return out.reshape(*pos.shape, d)
